# baseline (speedup 1.0000x reference)
import functools

import jax
import jax.numpy as jnp
from jax import lax
from jax.experimental import pallas as pl
from jax.experimental.pallas import tpu as pltpu
from jax.experimental.pallas import tpu_sc as plsc

NC = 2
NS = 16
NW = NC * NS
LANES = 16


def _round_up(v, m):
    return (v + m - 1) // m * m


def _mesh():
    return plsc.VectorSubcoreMesh(
        core_axis_name="c", subcore_axis_name="s", num_cores=NC, num_subcores=NS
    )


@functools.lru_cache(maxsize=None)
def _make_hist(e, np_):
    ept = e // NS
    half = np_ // NC
    accn = half + LANES

    @functools.partial(
        pl.kernel,
        out_type=jax.ShapeDtypeStruct((NS, np_), jnp.float32),
        mesh=_mesh(),
        scratch_types=[
            pltpu.VMEM((ept,), jnp.int32),
            pltpu.VMEM((accn,), jnp.float32),
            pltpu.SemaphoreType.DMA,
        ],
        compiler_params=pltpu.CompilerParams(needs_layout_passes=False),
    )
    def hist(edge_hbm, degp_hbm, idx_v, hist_v, sem):
        cid = lax.axis_index("c")
        sid = lax.axis_index("s")
        pltpu.sync_copy(edge_hbm.at[pl.ds(e + sid * ept, ept)], idx_v)

        def zero(i, c):
            hist_v[pl.ds(i * LANES, LANES)] = jnp.zeros((LANES,), jnp.float32)
            return c

        lax.fori_loop(0, accn // LANES, zero, 0, unroll=8)

        lo = pl.multiple_of(cid * half, 128)
        ones = jnp.ones((LANES,), jnp.float32)

        def body(i, c):
            d = idx_v[pl.ds(i * LANES, LANES)]
            rel = d - lo
            ok = (rel >= 0) & (rel < half)
            idx2 = jnp.where(ok, rel, half)
            plsc.addupdate_scatter(hist_v, [idx2], ones)
            return c

        lax.fori_loop(0, ept // LANES, body, 0, unroll=4)
        pltpu.sync_copy(
            hist_v.at[pl.ds(0, half)], degp_hbm.at[sid, pl.ds(lo, half)]
        )

    return hist


@functools.lru_cache(maxsize=None)
def _make_gather_scatter(e, np_):
    ept = e // NS
    half = np_ // NC
    accn = half + LANES
    sub = half // NS

    @functools.partial(
        pl.kernel,
        out_type=(
            jax.ShapeDtypeStruct((np_,), jnp.float32),
            jax.ShapeDtypeStruct((NW, half), jnp.float32),
        ),
        mesh=_mesh(),
        scratch_types=[
            pltpu.VMEM((ept,), jnp.int32),
            pltpu.VMEM((ept,), jnp.int32),
            pltpu.VMEM((np_,), jnp.float32),
            pltpu.VMEM((accn,), jnp.float32),
            pltpu.VMEM((NS, sub), jnp.float32),
            pltpu.VMEM((sub,), jnp.float32),
            pltpu.VMEM((sub,), jnp.float32),
            pltpu.VMEM((LANES,), jnp.float32),
            pltpu.SemaphoreType.DMA,
        ],
        compiler_params=pltpu.CompilerParams(needs_layout_passes=False),
    )
    def gs(edge_hbm, z_hbm, dinv_hbm, b_hbm, out_hbm, accs_hbm,
           src_v, dst_v, z_v, acc_v, col_v, dinv_v, out_v, b_v, sem):
        cid = lax.axis_index("c")
        sid = lax.axis_index("s")
        wid = cid * NS + sid
        lo = pl.multiple_of(cid * half, 128)
        own = pl.multiple_of(lo + sid * sub, 128)

        pltpu.sync_copy(edge_hbm.at[pl.ds(sid * ept, ept)], src_v)
        pltpu.sync_copy(edge_hbm.at[pl.ds(e + sid * ept, ept)], dst_v)
        pltpu.sync_copy(z_hbm, z_v)
        pltpu.sync_copy(dinv_hbm.at[pl.ds(own, sub)], dinv_v)
        pltpu.sync_copy(b_hbm, b_v)

        def zero(i, c):
            acc_v[pl.ds(i * LANES, LANES)] = jnp.zeros((LANES,), jnp.float32)
            return c

        lax.fori_loop(0, accn // LANES, zero, 0, unroll=8)

        def body(i, c):
            s = src_v[pl.ds(i * LANES, LANES)]
            d = dst_v[pl.ds(i * LANES, LANES)]
            v = plsc.load_gather(z_v, [s])
            rel = d - lo
            ok = (rel >= 0) & (rel < half)
            idx2 = jnp.where(ok, rel, half)
            plsc.addupdate_scatter(acc_v, [idx2], v)
            return c

        lax.fori_loop(0, ept // LANES, body, 0, unroll=4)

        pltpu.sync_copy(acc_v.at[pl.ds(0, half)], accs_hbm.at[wid])
        plsc.subcore_barrier()
        pltpu.sync_copy(
            accs_hbm.at[pl.ds(pl.multiple_of(cid * NS, 8), NS),
                        pl.ds(pl.multiple_of(sid * sub, 128), sub)],
            col_v,
        )

        def fold(j, c):
            acc16 = col_v[0, pl.ds(j * LANES, LANES)]
            for r in range(1, NS):
                acc16 = acc16 + col_v[r, pl.ds(j * LANES, LANES)]
            zj = z_v[pl.ds(own + j * LANES, LANES)]
            dj = dinv_v[pl.ds(j * LANES, LANES)]
            out_v[pl.ds(j * LANES, LANES)] = dj * (zj + acc16) + b_v[...]
            return c

        lax.fori_loop(0, sub // LANES, fold, 0, unroll=2)
        pltpu.sync_copy(out_v, out_hbm.at[pl.ds(own, sub)])

    return gs


def _t1_body(x_ref, wt_ref, degp_ref, z_ref, dinv_ref):
    y = lax.dot_general(
        wt_ref[...], x_ref[...],
        dimension_numbers=(((1,), (1,)), ((), ())),
        preferred_element_type=jnp.float32,
    )
    deg = jnp.sum(degp_ref[...], axis=0) + 1.0
    dinv = lax.rsqrt(deg)
    dinv_ref[...] = dinv
    z_ref[...] = dinv * y[0]


def kernel(x, edge_index, W, b):
    n, d_in = x.shape
    e = edge_index.shape[1]
    BL = 2048
    np_ = _round_up(n, NC * NS * 128)
    gr = np_ // BL
    np0 = _round_up(n, BL)
    xg = np0 // BL - 1

    x_p = jnp.pad(x, ((0, np0 - n), (0, 0)))
    wt = W.T
    b16 = jnp.broadcast_to(b, (LANES,))

    ef = edge_index.reshape(2 * e)
    degp = _make_hist(e, np_)(ef)

    z, dinv = pl.pallas_call(
        _t1_body,
        grid=(gr,),
        in_specs=[
            pl.BlockSpec((BL, d_in), lambda i: (jnp.minimum(i, xg), 0)),
            pl.BlockSpec((1, d_in), lambda i: (0, 0)),
            pl.BlockSpec((NS, BL), lambda i: (0, i)),
        ],
        out_specs=[
            pl.BlockSpec((BL,), lambda i: (i,)),
            pl.BlockSpec((BL,), lambda i: (i,)),
        ],
        out_shape=[jax.ShapeDtypeStruct((np_,), jnp.float32)] * 2,
    )(x_p, wt, degp)

    out, _ = _make_gather_scatter(e, np_)(ef, z, dinv, b16)

    return out[:n].reshape(n, 1)


# speedup vs baseline: 1.1778x; 1.1778x over previous
import functools

import jax
import jax.numpy as jnp
from jax import lax
from jax.experimental import pallas as pl
from jax.experimental.pallas import tpu as pltpu
from jax.experimental.pallas import tpu_sc as plsc

NC = 2
NS = 16
NW = NC * NS
LANES = 16


def _round_up(v, m):
    return (v + m - 1) // m * m


def _mesh():
    return plsc.VectorSubcoreMesh(
        core_axis_name="c", subcore_axis_name="s", num_cores=NC, num_subcores=NS
    )


def _edge_split(e):
    esl = (e // (NS * 128)) * 128
    edma = e - esl * (NS - 1)
    return esl, edma


@functools.lru_cache(maxsize=None)
def _make_hist(e, np_):
    esl, edma = _edge_split(e)
    half = np_ // NC
    accn = half + LANES

    @functools.partial(
        pl.kernel,
        out_type=jax.ShapeDtypeStruct((NS, np_), jnp.float32),
        mesh=_mesh(),
        scratch_types=[
            pltpu.VMEM((2, edma), jnp.int32),
            pltpu.VMEM((accn,), jnp.float32),
            pltpu.SemaphoreType.DMA,
        ],
        compiler_params=pltpu.CompilerParams(needs_layout_passes=False),
    )
    def hist(edge_hbm, degp_hbm, ebuf, hist_v, sem):
        cid = lax.axis_index("c")
        sid = lax.axis_index("s")
        lo = pl.multiple_of(cid * half, 128)
        start = pl.multiple_of(sid * esl, 128)
        cp = pltpu.async_copy(
            edge_hbm.at[pl.ds(0, 2), pl.ds(start, edma)], ebuf, sem
        )

        def zero(i, c):
            hist_v[pl.ds(i * LANES, LANES)] = jnp.zeros((LANES,), jnp.float32)
            return c

        lax.fori_loop(0, accn // LANES, zero, 0, unroll=8)
        cp.wait()

        ones = jnp.ones((LANES,), jnp.float32)

        def body(i):
            d = ebuf[1, pl.ds(i, LANES)]
            rel = d - lo
            ok = (rel >= 0) & (rel < half)
            idx2 = jnp.where(ok, rel, half)
            plsc.addupdate_scatter(hist_v, [idx2], ones)

        plsc.parallel_loop(0, esl, LANES, unroll=8)(body)

        @pl.when(sid == NS - 1)
        def _tail():
            plsc.parallel_loop(esl, edma, LANES, unroll=1)(body)

        pltpu.sync_copy(
            hist_v.at[pl.ds(0, half)], degp_hbm.at[sid, pl.ds(lo, half)]
        )

    return hist


@functools.lru_cache(maxsize=None)
def _make_gather_scatter(e, np_):
    esl, edma = _edge_split(e)
    half = np_ // NC
    accn = half + LANES
    sub = half // NS

    @functools.partial(
        pl.kernel,
        out_type=(
            jax.ShapeDtypeStruct((np_,), jnp.float32),
            jax.ShapeDtypeStruct((NW, half), jnp.float32),
        ),
        mesh=_mesh(),
        scratch_types=[
            pltpu.VMEM((2, edma), jnp.int32),
            pltpu.VMEM((np_,), jnp.float32),
            pltpu.VMEM((accn,), jnp.float32),
            pltpu.VMEM((NS, sub), jnp.float32),
            pltpu.VMEM((sub,), jnp.float32),
            pltpu.VMEM((sub,), jnp.float32),
            pltpu.VMEM((sub,), jnp.float32),
            pltpu.SemaphoreType.DMA,
        ],
        compiler_params=pltpu.CompilerParams(needs_layout_passes=False),
    )
    def gs(edge_hbm, z_hbm, dinv_hbm, w_hbm, out_hbm, accs_hbm,
           ebuf, z_v, acc_v, col_v, dinv_v, w_v, out_v, sem):
        cid = lax.axis_index("c")
        sid = lax.axis_index("s")
        wid = cid * NS + sid
        lo = pl.multiple_of(cid * half, 128)
        own = pl.multiple_of(lo + sid * sub, 128)
        start = pl.multiple_of(sid * esl, 128)

        cp1 = pltpu.async_copy(
            edge_hbm.at[pl.ds(0, 2), pl.ds(start, edma)], ebuf, sem
        )
        cp2 = pltpu.async_copy(z_hbm, z_v, sem)
        cp3 = pltpu.async_copy(dinv_hbm.at[pl.ds(own, sub)], dinv_v, sem)
        cp4 = pltpu.async_copy(w_hbm.at[pl.ds(own, sub)], w_v, sem)

        def zero(i, c):
            acc_v[pl.ds(i * LANES, LANES)] = jnp.zeros((LANES,), jnp.float32)
            return c

        lax.fori_loop(0, accn // LANES, zero, 0, unroll=8)
        cp1.wait()
        cp2.wait()
        cp3.wait()
        cp4.wait()

        def body(i):
            s = ebuf[0, pl.ds(i, LANES)]
            d = ebuf[1, pl.ds(i, LANES)]
            v = plsc.load_gather(z_v, [s])
            rel = d - lo
            ok = (rel >= 0) & (rel < half)
            idx2 = jnp.where(ok, rel, half)
            plsc.addupdate_scatter(acc_v, [idx2], v)

        plsc.parallel_loop(0, esl, LANES, unroll=8)(body)

        @pl.when(sid == NS - 1)
        def _tail():
            plsc.parallel_loop(esl, edma, LANES, unroll=1)(body)

        pltpu.sync_copy(acc_v.at[pl.ds(0, half)], accs_hbm.at[wid])
        plsc.subcore_barrier()
        pltpu.sync_copy(
            accs_hbm.at[pl.ds(pl.multiple_of(cid * NS, 8), NS),
                        pl.ds(pl.multiple_of(sid * sub, 128), sub)],
            col_v,
        )

        def fold(j, c):
            acc16 = col_v[0, pl.ds(j * LANES, LANES)]
            for r in range(1, NS):
                acc16 = acc16 + col_v[r, pl.ds(j * LANES, LANES)]
            dj = dinv_v[pl.ds(j * LANES, LANES)]
            wj = w_v[pl.ds(j * LANES, LANES)]
            out_v[pl.ds(j * LANES, LANES)] = wj + dj * acc16
            return c

        lax.fori_loop(0, sub // LANES, fold, 0, unroll=2)
        pltpu.sync_copy(out_v, out_hbm.at[pl.ds(own, sub)])

    return gs


def _t1_body(x_ref, wt_ref, b_ref, degp_ref, z_ref, dinv_ref, w_ref):
    y = lax.dot_general(
        wt_ref[...], x_ref[...],
        dimension_numbers=(((1,), (1,)), ((), ())),
        preferred_element_type=jnp.float32,
    )
    deg = jnp.sum(degp_ref[...], axis=0) + 1.0
    dinv = lax.rsqrt(deg)
    z = dinv * y[0]
    dinv_ref[...] = dinv
    z_ref[...] = z
    w_ref[...] = dinv * z + b_ref[...]


def kernel(x, edge_index, W, b):
    n, d_in = x.shape
    e = edge_index.shape[1]
    BL = 2048
    np_ = _round_up(n, NC * NS * 128)
    gr = np_ // BL
    np0 = _round_up(n, BL)
    xg = np0 // BL - 1

    x_p = jnp.pad(x, ((0, np0 - n), (0, 0)))
    wt = W.T

    degp = _make_hist(e, np_)(edge_index)

    z, dinv, w = pl.pallas_call(
        _t1_body,
        grid=(gr,),
        in_specs=[
            pl.BlockSpec((BL, d_in), lambda i: (jnp.minimum(i, xg), 0)),
            pl.BlockSpec((1, d_in), lambda i: (0, 0)),
            pl.BlockSpec((1,), lambda i: (0,)),
            pl.BlockSpec((NS, BL), lambda i: (0, i)),
        ],
        out_specs=[
            pl.BlockSpec((BL,), lambda i: (i,)),
            pl.BlockSpec((BL,), lambda i: (i,)),
            pl.BlockSpec((BL,), lambda i: (i,)),
        ],
        out_shape=[jax.ShapeDtypeStruct((np_,), jnp.float32)] * 3,
    )(x_p, wt, b, degp)

    out, _ = _make_gather_scatter(e, np_)(edge_index, z, dinv, w)

    return out[:n].reshape(n, 1)


# speedup vs baseline: 1.3383x; 1.1363x over previous
import functools

import jax
import jax.numpy as jnp
from jax import lax
from jax.experimental import pallas as pl
from jax.experimental.pallas import tpu as pltpu
from jax.experimental.pallas import tpu_sc as plsc

NC = 2
NS = 16
NW = NC * NS
LANES = 16


def _round_up(v, m):
    return (v + m - 1) // m * m


def _mesh():
    return plsc.VectorSubcoreMesh(
        core_axis_name="c", subcore_axis_name="s", num_cores=NC, num_subcores=NS
    )


def _edge_split(e):
    esl = (e // (NS * 128)) * 128
    edma = e - esl * (NS - 1)
    return esl, edma


def _rsqrt16(a):
    ii = plsc.bitcast(a, jnp.int32)
    r = plsc.bitcast(jnp.int32(0x5F3759DF) - (ii >> 1), jnp.float32)
    for _ in range(3):
        r = r * (1.5 - 0.5 * a * r * r)
    return r


@functools.lru_cache(maxsize=None)
def _make_sc(e, np_):
    esl, edma = _edge_split(e)
    half = np_ // NC
    accn = half + LANES
    sub = half // NS
    fch = np_ // NS

    @functools.partial(
        pl.kernel,
        out_type=(
            jax.ShapeDtypeStruct((np_,), jnp.float32),
            jax.ShapeDtypeStruct((NC, NS, np_), jnp.float32),
            jax.ShapeDtypeStruct((np_,), jnp.float32),
            jax.ShapeDtypeStruct((np_,), jnp.float32),
            jax.ShapeDtypeStruct((np_,), jnp.float32),
            jax.ShapeDtypeStruct((NC, NS, half), jnp.float32),
        ),
        mesh=_mesh(),
        scratch_types=[
            pltpu.VMEM((2, edma), jnp.int32),
            pltpu.VMEM((np_,), jnp.float32),
            pltpu.VMEM((NS, fch), jnp.float32),
            pltpu.VMEM((fch,), jnp.float32),
            pltpu.VMEM((fch,), jnp.float32),
            pltpu.VMEM((fch,), jnp.float32),
            pltpu.VMEM((fch,), jnp.float32),
            pltpu.VMEM((np_,), jnp.float32),
            pltpu.VMEM((accn,), jnp.float32),
            pltpu.VMEM((NS, sub), jnp.float32),
            pltpu.VMEM((sub,), jnp.float32),
            pltpu.VMEM((sub,), jnp.float32),
            pltpu.VMEM((sub,), jnp.float32),
            pltpu.VMEM((LANES,), jnp.float32),
            pltpu.SemaphoreType.DMA,
        ],
        compiler_params=pltpu.CompilerParams(needs_layout_passes=False),
    )
    def sck(edge_hbm, y_hbm, b_hbm,
            out_hbm, hstage, zs, dvs, ws, astage,
            ebuf, hist_v, colh, ych, zch, dch, wch, z_v, acc_v, cola,
            dinv_v, w_v, out_v, b_v, sem):
        cid = lax.axis_index("c")
        sid = lax.axis_index("s")
        lo = pl.multiple_of(cid * half, 128)
        own = pl.multiple_of(lo + sid * sub, 128)
        start = pl.multiple_of(sid * esl, 128)
        fstart = pl.multiple_of(sid * fch, 128)

        cp1 = pltpu.async_copy(
            edge_hbm.at[pl.ds(0, 2), pl.ds(start, edma)], ebuf, sem
        )
        cp2 = pltpu.async_copy(y_hbm.at[pl.ds(fstart, fch)], ych, sem)
        cp3 = pltpu.async_copy(b_hbm, b_v, sem)

        def zero_hist(i, c):
            hist_v[pl.ds(i * LANES, LANES)] = jnp.zeros((LANES,), jnp.float32)
            return c

        lax.fori_loop(0, np_ // LANES, zero_hist, 0, unroll=8)
        cp1.wait()
        cp2.wait()
        cp3.wait()

        ones = jnp.ones((LANES,), jnp.float32)

        def hbody(i):
            d = ebuf[1, pl.ds(i, LANES)]
            plsc.addupdate_scatter(hist_v, [d], ones)

        plsc.parallel_loop(0, esl, LANES, unroll=8)(hbody)

        @pl.when(sid == NS - 1)
        def _htail():
            plsc.parallel_loop(esl, edma, LANES, unroll=1)(hbody)

        pltpu.sync_copy(hist_v, hstage.at[cid, sid])
        plsc.subcore_barrier()
        pltpu.sync_copy(hstage.at[cid, pl.ds(0, NS), pl.ds(fstart, fch)], colh)

        def fold1(j, c):
            s16 = colh[0, pl.ds(j * LANES, LANES)]
            for r in range(1, NS):
                s16 = s16 + colh[r, pl.ds(j * LANES, LANES)]
            a = s16 + 1.0
            rinv = _rsqrt16(a)
            yj = ych[pl.ds(j * LANES, LANES)]
            zj = rinv * yj
            zch[pl.ds(j * LANES, LANES)] = zj
            dch[pl.ds(j * LANES, LANES)] = rinv
            wch[pl.ds(j * LANES, LANES)] = rinv * zj + b_v[...]
            return c

        lax.fori_loop(0, fch // LANES, fold1, 0, unroll=2)

        cp4 = pltpu.async_copy(zch, zs.at[pl.ds(fstart, fch)], sem)
        cp5 = pltpu.async_copy(dch, dvs.at[pl.ds(fstart, fch)], sem)
        cp6 = pltpu.async_copy(wch, ws.at[pl.ds(fstart, fch)], sem)

        def zero_acc(i, c):
            acc_v[pl.ds(i * LANES, LANES)] = jnp.zeros((LANES,), jnp.float32)
            return c

        lax.fori_loop(0, accn // LANES, zero_acc, 0, unroll=8)
        cp4.wait()
        cp5.wait()
        cp6.wait()
        plsc.subcore_barrier()

        cp7 = pltpu.async_copy(zs, z_v, sem)
        cp8 = pltpu.async_copy(dvs.at[pl.ds(own, sub)], dinv_v, sem)
        cp9 = pltpu.async_copy(ws.at[pl.ds(own, sub)], w_v, sem)
        cp7.wait()
        cp8.wait()
        cp9.wait()

        def gbody(i):
            s = ebuf[0, pl.ds(i, LANES)]
            d = ebuf[1, pl.ds(i, LANES)]
            v = plsc.load_gather(z_v, [s])
            rel = d - lo
            ok = (rel >= 0) & (rel < half)
            idx2 = jnp.where(ok, rel, half)
            plsc.addupdate_scatter(acc_v, [idx2], v)

        plsc.parallel_loop(0, esl, LANES, unroll=8)(gbody)

        @pl.when(sid == NS - 1)
        def _gtail():
            plsc.parallel_loop(esl, edma, LANES, unroll=1)(gbody)

        pltpu.sync_copy(acc_v.at[pl.ds(0, half)], astage.at[cid, sid])
        plsc.subcore_barrier()
        pltpu.sync_copy(
            astage.at[cid, pl.ds(0, NS),
                      pl.ds(pl.multiple_of(sid * sub, 128), sub)],
            cola,
        )

        def fold2(j, c):
            a16 = cola[0, pl.ds(j * LANES, LANES)]
            for r in range(1, NS):
                a16 = a16 + cola[r, pl.ds(j * LANES, LANES)]
            dj = dinv_v[pl.ds(j * LANES, LANES)]
            wj = w_v[pl.ds(j * LANES, LANES)]
            out_v[pl.ds(j * LANES, LANES)] = wj + dj * a16
            return c

        lax.fori_loop(0, sub // LANES, fold2, 0, unroll=2)
        pltpu.sync_copy(out_v, out_hbm.at[pl.ds(own, sub)])

    return sck


def _t1a_body(x_ref, wt_ref, y_ref):
    y = lax.dot_general(
        wt_ref[...], x_ref[...],
        dimension_numbers=(((1,), (1,)), ((), ())),
        preferred_element_type=jnp.float32,
    )
    y_ref[...] = y[0]


def kernel(x, edge_index, W, b):
    n, d_in = x.shape
    e = edge_index.shape[1]
    BL = 2048
    np_ = _round_up(n, NC * NS * 128)
    gr = np_ // BL
    xg = -(-n // BL) - 1

    wt = W.T
    b16 = jnp.broadcast_to(b, (LANES,))

    y = pl.pallas_call(
        _t1a_body,
        grid=(gr,),
        in_specs=[
            pl.BlockSpec((BL, d_in), lambda i: (jnp.minimum(i, xg), 0)),
            pl.BlockSpec((1, d_in), lambda i: (0, 0)),
        ],
        out_specs=pl.BlockSpec((BL,), lambda i: (i,)),
        out_shape=jax.ShapeDtypeStruct((np_,), jnp.float32),
    )(x, wt)

    out = _make_sc(e, np_)(edge_index, y, b16)[0]

    return out[:n].reshape(n, 1)


# speedup vs baseline: 1.3989x; 1.0453x over previous
import functools

import jax
import jax.numpy as jnp
from jax import lax
from jax.experimental import pallas as pl
from jax.experimental.pallas import tpu as pltpu
from jax.experimental.pallas import tpu_sc as plsc

NC = 2
NS = 16
NW = NC * NS
LANES = 16


def _round_up(v, m):
    return (v + m - 1) // m * m


def _mesh():
    return plsc.VectorSubcoreMesh(
        core_axis_name="c", subcore_axis_name="s", num_cores=NC, num_subcores=NS
    )


def _edge_split(e):
    esl = (e // (NS * 128)) * 128
    edma = e - esl * (NS - 1)
    return esl, edma


def _rsqrt16(a):
    ii = plsc.bitcast(a, jnp.int32)
    r = plsc.bitcast(jnp.int32(0x5F3759DF) - (ii >> 1), jnp.float32)
    for _ in range(3):
        r = r * (1.5 - 0.5 * a * r * r)
    return r


@functools.lru_cache(maxsize=None)
def _make_sc(e, np_, n):
    esl, edma = _edge_split(e)
    half = np_ // NC
    accn = half + LANES
    sub = half // NS
    fch = np_ // NS

    @functools.partial(
        pl.kernel,
        out_type=(
            jax.ShapeDtypeStruct((n,), jnp.float32),
            jax.ShapeDtypeStruct((NC, NS, np_), jnp.float32),
            jax.ShapeDtypeStruct((np_,), jnp.float32),
            jax.ShapeDtypeStruct((np_,), jnp.float32),
            jax.ShapeDtypeStruct((np_,), jnp.float32),
            jax.ShapeDtypeStruct((NC, NS, half), jnp.float32),
        ),
        mesh=_mesh(),
        scratch_types=[
            pltpu.VMEM((2, edma), jnp.int32),
            pltpu.VMEM((np_,), jnp.float32),
            pltpu.VMEM((NS, fch), jnp.float32),
            pltpu.VMEM((fch,), jnp.float32),
            pltpu.VMEM((fch,), jnp.float32),
            pltpu.VMEM((fch,), jnp.float32),
            pltpu.VMEM((fch,), jnp.float32),
            pltpu.VMEM((np_,), jnp.float32),
            pltpu.VMEM((accn,), jnp.float32),
            pltpu.VMEM((NS, sub), jnp.float32),
            pltpu.VMEM((sub,), jnp.float32),
            pltpu.VMEM((sub,), jnp.float32),
            pltpu.VMEM((sub,), jnp.float32),
            pltpu.VMEM((LANES,), jnp.float32),
            pltpu.SemaphoreType.DMA,
        ],
        compiler_params=pltpu.CompilerParams(needs_layout_passes=False),
    )
    def sck(edge_hbm, y_hbm, b_hbm,
            out_hbm, hstage, zs, dvs, ws, astage,
            ebuf, hist_v, colh, ych, zch, dch, wch, z_v, acc_v, cola,
            dinv_v, w_v, out_v, b_v, sem):
        cid = lax.axis_index("c")
        sid = lax.axis_index("s")
        lo = pl.multiple_of(cid * half, 128)
        own = pl.multiple_of(lo + sid * sub, 128)
        start = pl.multiple_of(sid * esl, 128)
        fstart = pl.multiple_of(sid * fch, 128)

        cp1 = pltpu.async_copy(
            edge_hbm.at[pl.ds(0, 2), pl.ds(start, edma)], ebuf, sem
        )
        cp2 = pltpu.async_copy(y_hbm.at[pl.ds(fstart, fch)], ych, sem)
        cp3 = pltpu.async_copy(b_hbm, b_v, sem)

        def zero_hist(i, c):
            hist_v[pl.ds(i * LANES, LANES)] = jnp.zeros((LANES,), jnp.float32)
            return c

        lax.fori_loop(0, np_ // LANES, zero_hist, 0, unroll=8)
        cp1.wait()
        cp2.wait()
        cp3.wait()

        ones = jnp.ones((LANES,), jnp.float32)

        def hbody(i):
            d = ebuf[1, pl.ds(i, LANES)]
            plsc.addupdate_scatter(hist_v, [d], ones)

        plsc.parallel_loop(0, esl, LANES, unroll=8)(hbody)

        @pl.when(sid == NS - 1)
        def _htail():
            plsc.parallel_loop(esl, edma, LANES, unroll=1)(hbody)

        pltpu.sync_copy(hist_v, hstage.at[cid, sid])
        plsc.subcore_barrier()
        pltpu.sync_copy(hstage.at[cid, pl.ds(0, NS), pl.ds(fstart, fch)], colh)

        def fold1(j, c):
            s16 = colh[0, pl.ds(j * LANES, LANES)]
            for r in range(1, NS):
                s16 = s16 + colh[r, pl.ds(j * LANES, LANES)]
            a = s16 + 1.0
            rinv = _rsqrt16(a)
            yj = ych[pl.ds(j * LANES, LANES)]
            zj = rinv * yj
            zch[pl.ds(j * LANES, LANES)] = zj
            dch[pl.ds(j * LANES, LANES)] = rinv
            wch[pl.ds(j * LANES, LANES)] = rinv * zj + b_v[...]
            return c

        lax.fori_loop(0, fch // LANES, fold1, 0, unroll=2)

        cp4 = pltpu.async_copy(zch, zs.at[pl.ds(fstart, fch)], sem)
        cp5 = pltpu.async_copy(dch, dvs.at[pl.ds(fstart, fch)], sem)
        cp6 = pltpu.async_copy(wch, ws.at[pl.ds(fstart, fch)], sem)

        def zero_acc(i, c):
            acc_v[pl.ds(i * LANES, LANES)] = jnp.zeros((LANES,), jnp.float32)
            return c

        lax.fori_loop(0, accn // LANES, zero_acc, 0, unroll=8)
        cp4.wait()
        cp5.wait()
        cp6.wait()
        plsc.subcore_barrier()

        cp7 = pltpu.async_copy(zs, z_v, sem)
        cp8 = pltpu.async_copy(dvs.at[pl.ds(own, sub)], dinv_v, sem)
        cp9 = pltpu.async_copy(ws.at[pl.ds(own, sub)], w_v, sem)
        cp7.wait()
        cp8.wait()
        cp9.wait()

        def gbody(i):
            s = ebuf[0, pl.ds(i, LANES)]
            d = ebuf[1, pl.ds(i, LANES)]
            v = plsc.load_gather(z_v, [s])
            rel = d - lo
            ok = (rel >= 0) & (rel < half)
            idx2 = jnp.where(ok, rel, half)
            plsc.addupdate_scatter(acc_v, [idx2], v)

        plsc.parallel_loop(0, esl, LANES, unroll=8)(gbody)

        @pl.when(sid == NS - 1)
        def _gtail():
            plsc.parallel_loop(esl, edma, LANES, unroll=1)(gbody)

        pltpu.sync_copy(acc_v.at[pl.ds(0, half)], astage.at[cid, sid])
        plsc.subcore_barrier()
        pltpu.sync_copy(
            astage.at[cid, pl.ds(0, NS),
                      pl.ds(pl.multiple_of(sid * sub, 128), sub)],
            cola,
        )

        def fold2(j, c):
            a16 = cola[0, pl.ds(j * LANES, LANES)]
            for r in range(1, NS):
                a16 = a16 + cola[r, pl.ds(j * LANES, LANES)]
            dj = dinv_v[pl.ds(j * LANES, LANES)]
            wj = w_v[pl.ds(j * LANES, LANES)]
            out_v[pl.ds(j * LANES, LANES)] = wj + dj * a16
            return c

        lax.fori_loop(0, sub // LANES, fold2, 0, unroll=2)

        @pl.when(own + sub <= n)
        def _full():
            pltpu.sync_copy(out_v, out_hbm.at[pl.ds(own, sub)])

        rem = n % sub
        if rem:
            @pl.when((own < n) & (own + sub > n))
            def _part():
                pltpu.sync_copy(
                    out_v.at[pl.ds(0, rem)], out_hbm.at[pl.ds(own, rem)]
                )

    return sck


def _t1a_body(x_ref, wt_ref, b_ref, y_ref, b16_ref):
    y = lax.dot_general(
        wt_ref[...], x_ref[...],
        dimension_numbers=(((1,), (1,)), ((), ())),
        preferred_element_type=jnp.float32,
    )
    y_ref[...] = y[0]
    b16_ref[...] = jnp.broadcast_to(b_ref[...], (LANES,))


def kernel(x, edge_index, W, b):
    n, d_in = x.shape
    e = edge_index.shape[1]
    BL = 4096
    np_ = _round_up(n, NC * NS * 128)
    gr = np_ // BL
    xg = -(-n // BL) - 1

    wt = W.T

    y, b16 = pl.pallas_call(
        _t1a_body,
        grid=(gr,),
        in_specs=[
            pl.BlockSpec((BL, d_in), lambda i: (jnp.minimum(i, xg), 0)),
            pl.BlockSpec((1, d_in), lambda i: (0, 0)),
            pl.BlockSpec((1,), lambda i: (0,)),
        ],
        out_specs=[
            pl.BlockSpec((BL,), lambda i: (i,)),
            pl.BlockSpec((LANES,), lambda i: (0,)),
        ],
        out_shape=[
            jax.ShapeDtypeStruct((np_,), jnp.float32),
            jax.ShapeDtypeStruct((LANES,), jnp.float32),
        ],
    )(x, wt, b)

    out = _make_sc(e, np_, n)(edge_index, y, b16)[0]

    return out.reshape(n, 1)
